# Initial kernel scaffold; baseline (speedup 1.0000x reference)
import jax
import jax.numpy as jnp
from jax.experimental import pallas as pl


def kernel(x_cont, x_cat, edge_index, edge_attr, batch, norm, tab_chrg, tab_pdg, W_cont, b_cont, W_cat, b_cat, W_enc, b_enc, g_all, b_all, W_msg0, b_msg0, g_bn0, b_bn0, W_msg1, b_msg1, g_bn1, b_bn1, W_out1, b_out1, W_out2, b_out2):
    raise NotImplementedError("write your pallas kernel here")


# speedup vs baseline: 2.9140x; 2.9140x over previous
import functools

import jax
import jax.numpy as jnp
from jax import lax
from jax.experimental import pallas as pl
from jax.experimental.pallas import tpu as pltpu
from jax.experimental.pallas import tpu_sc as plsc

N = 50000
E = 800000
HID = 64
PDG_LIST = [1, 2, 11, 13, 22, 130, 211]

NC = 2
NS = 16
HALF = N // NC
HALF_PAD = 25088
RPS = HALF_PAD // NS
DUMMY = 25024
CH = 128
E_PAD = 819200
CPS = E_PAD // NS // CH

RB = 2000
GRID = N // RB

def _elu(x):
    return jnp.where(x > 0, x, jnp.exp(jnp.minimum(x, 0.0)) - 1.0)


def _clamp_cols(colv, idxv, nb):
    def body(j, _):
        off = pl.multiple_of(j * 16, 16)
        cv = colv[pl.ds(off, 16)]
        lc = cv - nb
        ok = (lc >= 0) & (lc < HALF)
        idxv[pl.ds(off, 16)] = jnp.where(ok, lc, DUMMY)
        return 0
    lax.fori_loop(0, CH // 16, body, 0)


@functools.cache
def _sc_kernels():
    mesh = plsc.VectorSubcoreMesh(core_axis_name="c", subcore_axis_name="s",
                                  num_cores=NC, num_subcores=NS)

    @functools.partial(
        pl.kernel, mesh=mesh,
        out_type=jax.ShapeDtypeStruct((NC * HALF_PAD, 16), jnp.float32),
        scratch_types=[
            pltpu.VMEM((CH,), jnp.int32),
            pltpu.VMEM((CH,), jnp.int32),
            pltpu.VMEM((CH, 16), jnp.float32),
            pltpu.VMEM_SHARED((HALF_PAD, 16), jnp.float32),
        ],
        compiler_params=pltpu.CompilerParams(use_tc_tiling_on_sc=False),
    )
    def sc_count(col_hbm, ones_hbm, zero_hbm, cnt_hbm, colv, idxv, onesv, acc):
        c = lax.axis_index("c")
        s = lax.axis_index("s")
        nb = c * HALF
        pltpu.sync_copy(zero_hbm.at[pl.ds(s * RPS, RPS)],
                        acc.at[pl.ds(s * RPS, RPS)])
        pltpu.sync_copy(ones_hbm, onesv)
        plsc.subcore_barrier()

        def chunk(t, _):
            base = s * (E_PAD // NS) + t * CH
            pltpu.sync_copy(col_hbm.at[pl.ds(base, CH)], colv)
            _clamp_cols(colv, idxv, nb)
            pltpu.sync_copy(onesv, acc.at[idxv], add=True)
            return 0

        lax.fori_loop(0, CPS, chunk, 0)
        plsc.subcore_barrier()
        pltpu.sync_copy(acc.at[pl.ds(s * RPS, RPS)],
                        cnt_hbm.at[pl.ds(c * HALF_PAD + s * RPS, RPS)])

    @functools.partial(
        pl.kernel, mesh=mesh,
        out_type=jax.ShapeDtypeStruct((NC * HALF_PAD, HID), jnp.float32),
        scratch_types=[
            pltpu.VMEM((CH,), jnp.int32),
            pltpu.VMEM((CH,), jnp.int32),
            pltpu.VMEM((CH,), jnp.int32),
            pltpu.VMEM((CH, HID), jnp.float32),
            pltpu.VMEM_SHARED((HALF_PAD, HID), jnp.float32),
            pltpu.SemaphoreType.DMA,
        ],
        compiler_params=pltpu.CompilerParams(use_tc_tiling_on_sc=False),
    )
    def sc_segsum(a_hbm, row_hbm, col_hbm, zero_hbm, out_hbm,
                  rowv, colv, idxv, gbuf, acc, sem):
        c = lax.axis_index("c")
        s = lax.axis_index("s")
        nb = c * HALF
        pltpu.sync_copy(zero_hbm.at[pl.ds(s * RPS, RPS)],
                        acc.at[pl.ds(s * RPS, RPS)])
        plsc.subcore_barrier()

        def chunk(t, _):
            base = s * (E_PAD // NS) + t * CH
            pltpu.sync_copy(row_hbm.at[pl.ds(base, CH)], rowv)
            pltpu.sync_copy(col_hbm.at[pl.ds(base, CH)], colv)
            _clamp_cols(colv, idxv, nb)
            pltpu.async_copy(a_hbm.at[rowv], gbuf, sem).wait()
            pltpu.sync_copy(gbuf, acc.at[idxv], add=True)
            return 0

        lax.fori_loop(0, CPS, chunk, 0)
        plsc.subcore_barrier()
        pltpu.sync_copy(acc.at[pl.ds(s * RPS, RPS)],
                        out_hbm.at[pl.ds(c * HALF_PAD + s * RPS, RPS)])

    return sc_count, sc_segsum


def _sc_count(col, ones16, zero16):
    return _sc_kernels()[0](col, ones16, zero16)


def _sc_segsum(a, row, col, zero64):
    return _sc_kernels()[1](a, row, col, zero64)


def _row_spec(cols):
    return pl.BlockSpec((RB, cols), lambda i: (i, 0))


def _full_spec(shape):
    return pl.BlockSpec(shape, lambda i: tuple(0 for _ in shape))


def _acc_sums(i, x, s_ref):
    @pl.when(i == 0)
    def _():
        s_ref[...] = jnp.zeros_like(s_ref)
    s1 = jnp.sum(x, axis=0, keepdims=True)
    s2 = jnp.sum(x * x, axis=0, keepdims=True)
    s_ref[...] += jnp.concatenate([s1, s2], axis=0)


def _enc_body(xc_ref, xcat_ref, norm_ref, tc_ref, tp_ref, wc_ref, bc_ref,
              wk_ref, bk_ref, we_ref, be_ref, z_ref, s_ref):
    i = pl.program_id(0)
    xc = xc_ref[...] * norm_ref[...]
    emb_cont = _elu(jnp.dot(xc, wc_ref[...], preferred_element_type=jnp.float32)
                    + bc_ref[...])
    chrg = xcat_ref[:, 1:2] + 1
    oh_c = (chrg == lax.broadcasted_iota(jnp.int32, (RB, 3), 1)).astype(jnp.float32)
    emb_chrg = jnp.dot(oh_c, tc_ref[...], preferred_element_type=jnp.float32)
    r = jnp.abs(xcat_ref[:, 0:1])
    for k, p in enumerate(PDG_LIST):
        r = jnp.where(r == p, jnp.full_like(r, k), r)
    oh_p = (r == lax.broadcasted_iota(jnp.int32, (RB, 7), 1)).astype(jnp.float32)
    emb_pdg = jnp.dot(oh_p, tp_ref[...], preferred_element_type=jnp.float32)
    cat = jnp.concatenate([emb_chrg, emb_pdg], axis=1)
    emb_cat = _elu(jnp.dot(cat, wk_ref[...], preferred_element_type=jnp.float32)
                   + bk_ref[...])
    both = jnp.concatenate([emb_cat, emb_cont], axis=1)
    z = _elu(jnp.dot(both, we_ref[...], preferred_element_type=jnp.float32)
             + be_ref[...])
    z_ref[...] = z
    _acc_sums(i, z, s_ref)


_enc_call = pl.pallas_call(
    _enc_body,
    grid=(GRID,),
    in_specs=[
        _row_spec(6), _row_spec(2), _full_spec((1, 6)),
        _full_spec((3, 16)), _full_spec((7, 16)),
        _full_spec((6, 32)), _full_spec((1, 32)),
        _full_spec((32, 32)), _full_spec((1, 32)),
        _full_spec((64, 64)), _full_spec((1, 64)),
    ],
    out_specs=[_row_spec(64), _full_spec((2, 64))],
    out_shape=[jax.ShapeDtypeStruct((N, 64), jnp.float32),
               jax.ShapeDtypeStruct((2, 64), jnp.float32)],
)


def _bn(x, m_ref, v_ref, g_ref, b_ref):
    return (g_ref[...] * (x - m_ref[...]) * lax.rsqrt(v_ref[...] + 1e-5)
            + b_ref[...])


def _proj_body(x_ref, m_ref, v_ref, g_ref, b_ref, wt_ref, wb_ref, bm_ref,
               emb_ref, a_ref, bb_ref):
    emb = _bn(x_ref[...], m_ref, v_ref, g_ref, b_ref)
    emb_ref[...] = emb
    a_ref[...] = jnp.dot(emb, wt_ref[...], preferred_element_type=jnp.float32)
    bb_ref[...] = (jnp.dot(emb, wb_ref[...], preferred_element_type=jnp.float32)
                   + bm_ref[...])


def _proj_res_body(x_ref, base_ref, m_ref, v_ref, g_ref, b_ref, wt_ref, wb_ref,
                   bm_ref, emb_ref, a_ref, bb_ref):
    emb = base_ref[...] + _bn(x_ref[...], m_ref, v_ref, g_ref, b_ref)
    emb_ref[...] = emb
    a_ref[...] = jnp.dot(emb, wt_ref[...], preferred_element_type=jnp.float32)
    bb_ref[...] = (jnp.dot(emb, wb_ref[...], preferred_element_type=jnp.float32)
                   + bm_ref[...])


_proj_specs = dict(
    grid=(GRID,),
    out_specs=[_row_spec(64), _row_spec(64), _row_spec(64)],
    out_shape=[jax.ShapeDtypeStruct((N, 64), jnp.float32)] * 3,
)
_proj_call = pl.pallas_call(
    _proj_body,
    in_specs=[_row_spec(64), _full_spec((1, 64)), _full_spec((1, 64)),
              _full_spec((1, 64)), _full_spec((1, 64)),
              _full_spec((64, 64)), _full_spec((64, 64)), _full_spec((1, 64))],
    **_proj_specs,
)
_proj_res_call = pl.pallas_call(
    _proj_res_body,
    in_specs=[_row_spec(64), _row_spec(64), _full_spec((1, 64)),
              _full_spec((1, 64)), _full_spec((1, 64)), _full_spec((1, 64)),
              _full_spec((64, 64)), _full_spec((64, 64)), _full_spec((1, 64))],
    **_proj_specs,
)


def _combine_body(s_ref, cnt_ref, bb_ref, o_ref, sum_ref):
    i = pl.program_id(0)
    cnt = cnt_ref[...]
    o = jnp.where(cnt > 0,
                  s_ref[...] / jnp.maximum(cnt, 1.0) + bb_ref[...],
                  0.0)
    o_ref[...] = o
    _acc_sums(i, o, sum_ref)


_combine_call = pl.pallas_call(
    _combine_body,
    grid=(GRID,),
    in_specs=[_row_spec(64), _row_spec(1), _row_spec(64)],
    out_specs=[_row_spec(64), _full_spec((2, 64))],
    out_shape=[jax.ShapeDtypeStruct((N, 64), jnp.float32),
               jax.ShapeDtypeStruct((2, 64), jnp.float32)],
)


def _final_body(x_ref, base_ref, m_ref, v_ref, g_ref, b_ref, w1_ref, b1_ref,
                w2_ref, b2_ref, y_ref):
    emb = base_ref[...] + _bn(x_ref[...], m_ref, v_ref, g_ref, b_ref)
    h = _elu(jnp.dot(emb, w1_ref[...], preferred_element_type=jnp.float32)
             + b1_ref[...])
    y_ref[...] = (jnp.dot(h, w2_ref[...], preferred_element_type=jnp.float32)
                  + b2_ref[...])


_final_call = pl.pallas_call(
    _final_body,
    grid=(GRID,),
    in_specs=[_row_spec(64), _row_spec(64), _full_spec((1, 64)),
              _full_spec((1, 64)), _full_spec((1, 64)), _full_spec((1, 64)),
              _full_spec((64, 32)), _full_spec((1, 32)),
              _full_spec((32, 1)), _full_spec((1, 1))],
    out_specs=[_row_spec(1)],
    out_shape=[jax.ShapeDtypeStruct((N, 1), jnp.float32)],
)


def _moments(s):
    m = s[0:1] / N
    v = s[1:2] / N - m * m
    return m, v


def _halves(x):
    return jnp.concatenate([x[:HALF], x[HALF_PAD:HALF_PAD + HALF]], axis=0)


def kernel(x_cont, x_cat, edge_index, edge_attr, batch, norm, tab_chrg,
           tab_pdg, W_cont, b_cont, W_cat, b_cat, W_enc, b_enc, g_all, b_all,
           W_msg0, b_msg0, g_bn0, b_bn0, W_msg1, b_msg1, g_bn1, b_bn1,
           W_out1, b_out1, W_out2, b_out2):
    row = jnp.concatenate([edge_index[0], jnp.zeros((E_PAD - E,), jnp.int32)])
    col = jnp.concatenate([edge_index[1], jnp.full((E_PAD - E,), N, jnp.int32)])
    ones16 = jnp.ones((CH, 16), jnp.float32)
    zero16 = jnp.zeros((HALF_PAD, 16), jnp.float32)
    zero64 = jnp.zeros((HALF_PAD, HID), jnp.float32)

    cnt_pad = _sc_count(col, ones16, zero16)
    cnt = _halves(cnt_pad)[:, 0:1]

    z, s = _enc_call(x_cont, x_cat, norm.reshape(1, 6), tab_chrg, tab_pdg,
                     W_cont, b_cont.reshape(1, 32), W_cat, b_cat.reshape(1, 32),
                     W_enc, b_enc.reshape(1, 64))
    m, v = _moments(s)
    emb, a0, bb0 = _proj_call(z, m, v, g_all.reshape(1, 64),
                              b_all.reshape(1, 64), W_msg0[:HID], W_msg0[HID:],
                              b_msg0.reshape(1, 64))
    s0 = _halves(_sc_segsum(a0, row, col, zero64))
    out0, s = _combine_call(s0, cnt, bb0)
    m, v = _moments(s)
    emb1, a1, bb1 = _proj_res_call(out0, emb, m, v, g_bn0.reshape(1, 64),
                                   b_bn0.reshape(1, 64), W_msg1[:HID],
                                   W_msg1[HID:], b_msg1.reshape(1, 64))
    s1 = _halves(_sc_segsum(a1, row, col, zero64))
    out1, s = _combine_call(s1, cnt, bb1)
    m, v = _moments(s)
    (y,) = _final_call(out1, emb1, m, v, g_bn1.reshape(1, 64),
                       b_bn1.reshape(1, 64), W_out1, b_out1.reshape(1, 32),
                       W_out2, b_out2.reshape(1, 1))
    return y[:, 0]


# speedup vs baseline: 6.8328x; 2.3448x over previous
import functools

import jax
import jax.numpy as jnp
from jax import lax
from jax.experimental import pallas as pl
from jax.experimental.pallas import tpu as pltpu
from jax.experimental.pallas import tpu_sc as plsc

N = 50000
E = 800000
HID = 64
PDG_LIST = [1, 2, 11, 13, 22, 130, 211]

NC = 2
NS = 16
CH = 128
E_PAD = 819200
CR = E_PAD // CH
NPAD = 50048
RPN = NPAD // NS
G = 2
NSC = CR // NS // G
G2 = 10
NSC2 = CR // (NC * NS) // G2

RB = 2000
GRID = N // RB

def _elu(x):
    return jnp.where(x > 0, x, jnp.exp(jnp.minimum(x, 0.0)) - 1.0)


@functools.cache
def _sc_kernels():
    mesh = plsc.VectorSubcoreMesh(core_axis_name="c", subcore_axis_name="s",
                                  num_cores=NC, num_subcores=NS)

    @functools.partial(
        pl.kernel, mesh=mesh,
        out_type=jax.ShapeDtypeStruct((NC * NPAD, 16), jnp.float32),
        scratch_types=[
            pltpu.VMEM((2, G2, CH), jnp.int32),
            pltpu.VMEM((CH, 16), jnp.float32),
            pltpu.VMEM_SHARED((NPAD, 16), jnp.float32),
            pltpu.SemaphoreType.DMA,
            pltpu.SemaphoreType.DMA,
            pltpu.SemaphoreType.DMA,
            pltpu.SemaphoreType.DMA,
        ],
        compiler_params=pltpu.CompilerParams(use_tc_tiling_on_sc=False),
    )
    def sc_count(col_hbm, ones_hbm, zero_hbm, cnt_hbm, colb, onesv, acc,
                 isem0, isem1, ssem0, ssem1):
        c = lax.axis_index("c")
        s = lax.axis_index("s")
        isem = (isem0, isem1)
        ssem = (ssem0, ssem1)
        sub = (c * NS + s) * (G2 * NSC2)
        pltpu.sync_copy(zero_hbm.at[pl.ds(s * RPN, RPN)],
                        acc.at[pl.ds(s * RPN, RPN)])
        pltpu.sync_copy(ones_hbm, onesv)
        plsc.subcore_barrier()

        def load(u, b):
            pltpu.async_copy(col_hbm.at[pl.ds(sub + u * G2, G2)],
                             colb.at[b], isem[b])

        def wait_load(b):
            pltpu.make_async_copy(col_hbm.at[pl.ds(sub, G2)],
                                  colb.at[b], isem[b]).wait()

        def fire_scatter(b):
            for j in range(G2):
                pltpu.async_copy(onesv, acc.at[colb.at[b, j]], ssem[b],
                                 add=True)

        def wait_scatter(b):
            for j in range(G2):
                pltpu.make_async_copy(onesv, acc.at[colb.at[b, j]],
                                      ssem[b]).wait()

        load(0, 0)
        load(1, 1)

        def outer(w, _):
            for off, b in ((0, 0), (1, 1)):
                u = 2 * w + off
                wait_load(b)
                fire_scatter(b)
                wait_scatter(b)

                @pl.when(u + 2 < NSC2)
                def _():
                    load(u + 2, b)
            return 0

        lax.fori_loop(0, NSC2 // 2, outer, 0)
        plsc.subcore_barrier()
        pltpu.sync_copy(acc.at[pl.ds(s * RPN, RPN)],
                        cnt_hbm.at[pl.ds(c * NPAD + s * RPN, RPN)])

    @functools.partial(
        pl.kernel, mesh=mesh,
        out_type=jax.ShapeDtypeStruct((NC * NPAD, HID // NC), jnp.float32),
        scratch_types=[
            pltpu.VMEM((2, G, CH), jnp.int32),
            pltpu.VMEM((2, G, CH), jnp.int32),
            pltpu.VMEM((2, G, CH, HID // NC), jnp.float32),
            pltpu.VMEM_SHARED((NPAD, HID // NC), jnp.float32),
            pltpu.SemaphoreType.DMA,
            pltpu.SemaphoreType.DMA,
            pltpu.SemaphoreType.DMA,
            pltpu.SemaphoreType.DMA,
            pltpu.SemaphoreType.DMA,
            pltpu.SemaphoreType.DMA,
        ],
        compiler_params=pltpu.CompilerParams(use_tc_tiling_on_sc=False),
    )
    def sc_segsum(a_hbm, rows_hbm, col_hbm, zero_hbm, out_hbm,
                  rowb, colb, gb, acc,
                  isem0, isem1, gsem0, gsem1, ssem0, ssem1):
        c = lax.axis_index("c")
        s = lax.axis_index("s")
        isem = (isem0, isem1)
        gsem = (gsem0, gsem1)
        ssem = (ssem0, ssem1)
        sub = s * (G * NSC)
        rbase = c * CR + sub
        pltpu.sync_copy(zero_hbm.at[pl.ds(s * RPN, RPN)],
                        acc.at[pl.ds(s * RPN, RPN)])
        plsc.subcore_barrier()

        def load(u, b):
            pltpu.async_copy(rows_hbm.at[pl.ds(rbase + u * G, G)],
                             rowb.at[b], isem[b])
            pltpu.async_copy(col_hbm.at[pl.ds(sub + u * G, G)],
                             colb.at[b], isem[b])

        def wait_load(b):
            pltpu.make_async_copy(rows_hbm.at[pl.ds(rbase, G)],
                                  rowb.at[b], isem[b]).wait()
            pltpu.make_async_copy(col_hbm.at[pl.ds(sub, G)],
                                  colb.at[b], isem[b]).wait()

        def fire_gather(b):
            for j in range(G):
                pltpu.async_copy(a_hbm.at[rowb.at[b, j]], gb.at[b, j],
                                 gsem[b])

        def wait_gather(b):
            for j in range(G):
                pltpu.make_async_copy(a_hbm.at[rowb.at[b, j]], gb.at[b, j],
                                      gsem[b]).wait()

        def fire_scatter(b):
            for j in range(G):
                pltpu.async_copy(gb.at[b, j], acc.at[colb.at[b, j]], ssem[b],
                                 add=True)

        def wait_scatter(b):
            for j in range(G):
                pltpu.make_async_copy(gb.at[b, j], acc.at[colb.at[b, j]],
                                      ssem[b]).wait()

        load(0, 0)
        wait_load(0)
        fire_gather(0)
        load(1, 1)

        def outer(w, _):
            for off, b, nb in ((0, 0, 1), (1, 1, 0)):
                u = 2 * w + off

                @pl.when(u + 1 < NSC)
                def _():
                    wait_load(nb)
                    fire_gather(nb)

                wait_gather(b)
                fire_scatter(b)
                wait_scatter(b)

                @pl.when(u + 2 < NSC)
                def _():
                    load(u + 2, b)
            return 0

        lax.fori_loop(0, NSC // 2, outer, 0)
        plsc.subcore_barrier()
        pltpu.sync_copy(acc.at[pl.ds(s * RPN, RPN)],
                        out_hbm.at[pl.ds(c * NPAD + s * RPN, RPN)])

    return sc_count, sc_segsum


def _sc_count(col2d, ones16, zero16):
    return _sc_kernels()[0](col2d, ones16, zero16)


def _sc_segsum(a_st, rows_st, col2d, zero32):
    return _sc_kernels()[1](a_st, rows_st, col2d, zero32)


def _row_spec(cols):
    return pl.BlockSpec((RB, cols), lambda i: (i, 0))


def _full_spec(shape):
    return pl.BlockSpec(shape, lambda i: tuple(0 for _ in shape))


def _acc_sums(i, x, s_ref):
    @pl.when(i == 0)
    def _():
        s_ref[...] = jnp.zeros_like(s_ref)
    s1 = jnp.sum(x, axis=0, keepdims=True)
    s2 = jnp.sum(x * x, axis=0, keepdims=True)
    s_ref[...] += jnp.concatenate([s1, s2], axis=0)


def _enc_body(xc_ref, xcat_ref, norm_ref, tc_ref, tp_ref, wc_ref, bc_ref,
              wk_ref, bk_ref, we_ref, be_ref, z_ref, s_ref):
    i = pl.program_id(0)
    xc = xc_ref[...] * norm_ref[...]
    emb_cont = _elu(jnp.dot(xc, wc_ref[...], preferred_element_type=jnp.float32)
                    + bc_ref[...])
    chrg = xcat_ref[:, 1:2] + 1
    oh_c = (chrg == lax.broadcasted_iota(jnp.int32, (RB, 3), 1)).astype(jnp.float32)
    emb_chrg = jnp.dot(oh_c, tc_ref[...], preferred_element_type=jnp.float32)
    r = jnp.abs(xcat_ref[:, 0:1])
    for k, p in enumerate(PDG_LIST):
        r = jnp.where(r == p, jnp.full_like(r, k), r)
    oh_p = (r == lax.broadcasted_iota(jnp.int32, (RB, 7), 1)).astype(jnp.float32)
    emb_pdg = jnp.dot(oh_p, tp_ref[...], preferred_element_type=jnp.float32)
    cat = jnp.concatenate([emb_chrg, emb_pdg], axis=1)
    emb_cat = _elu(jnp.dot(cat, wk_ref[...], preferred_element_type=jnp.float32)
                   + bk_ref[...])
    both = jnp.concatenate([emb_cat, emb_cont], axis=1)
    z = _elu(jnp.dot(both, we_ref[...], preferred_element_type=jnp.float32)
             + be_ref[...])
    z_ref[...] = z
    _acc_sums(i, z, s_ref)


_enc_call = pl.pallas_call(
    _enc_body,
    grid=(GRID,),
    in_specs=[
        _row_spec(6), _row_spec(2), _full_spec((1, 6)),
        _full_spec((3, 16)), _full_spec((7, 16)),
        _full_spec((6, 32)), _full_spec((1, 32)),
        _full_spec((32, 32)), _full_spec((1, 32)),
        _full_spec((64, 64)), _full_spec((1, 64)),
    ],
    out_specs=[_row_spec(64), _full_spec((2, 64))],
    out_shape=[jax.ShapeDtypeStruct((N, 64), jnp.float32),
               jax.ShapeDtypeStruct((2, 64), jnp.float32)],
)


def _bn(x, m_ref, v_ref, g_ref, b_ref):
    return (g_ref[...] * (x - m_ref[...]) * lax.rsqrt(v_ref[...] + 1e-5)
            + b_ref[...])


def _proj_body(x_ref, m_ref, v_ref, g_ref, b_ref, wt_ref, wb_ref, bm_ref,
               emb_ref, a_ref, bb_ref):
    emb = _bn(x_ref[...], m_ref, v_ref, g_ref, b_ref)
    emb_ref[...] = emb
    a_ref[...] = jnp.dot(emb, wt_ref[...], preferred_element_type=jnp.float32)
    bb_ref[...] = (jnp.dot(emb, wb_ref[...], preferred_element_type=jnp.float32)
                   + bm_ref[...])


def _proj_res_body(x_ref, base_ref, m_ref, v_ref, g_ref, b_ref, wt_ref, wb_ref,
                   bm_ref, emb_ref, a_ref, bb_ref):
    emb = base_ref[...] + _bn(x_ref[...], m_ref, v_ref, g_ref, b_ref)
    emb_ref[...] = emb
    a_ref[...] = jnp.dot(emb, wt_ref[...], preferred_element_type=jnp.float32)
    bb_ref[...] = (jnp.dot(emb, wb_ref[...], preferred_element_type=jnp.float32)
                   + bm_ref[...])


_proj_specs = dict(
    grid=(GRID,),
    out_specs=[_row_spec(64), _row_spec(64), _row_spec(64)],
    out_shape=[jax.ShapeDtypeStruct((N, 64), jnp.float32)] * 3,
)
_proj_call = pl.pallas_call(
    _proj_body,
    in_specs=[_row_spec(64), _full_spec((1, 64)), _full_spec((1, 64)),
              _full_spec((1, 64)), _full_spec((1, 64)),
              _full_spec((64, 64)), _full_spec((64, 64)), _full_spec((1, 64))],
    **_proj_specs,
)
_proj_res_call = pl.pallas_call(
    _proj_res_body,
    in_specs=[_row_spec(64), _row_spec(64), _full_spec((1, 64)),
              _full_spec((1, 64)), _full_spec((1, 64)), _full_spec((1, 64)),
              _full_spec((64, 64)), _full_spec((64, 64)), _full_spec((1, 64))],
    **_proj_specs,
)


def _combine_body(s_ref, cnt_ref, bb_ref, o_ref, sum_ref):
    i = pl.program_id(0)
    cnt = cnt_ref[...]
    o = jnp.where(cnt > 0,
                  s_ref[...] / jnp.maximum(cnt, 1.0) + bb_ref[...],
                  0.0)
    o_ref[...] = o
    _acc_sums(i, o, sum_ref)


_combine_call = pl.pallas_call(
    _combine_body,
    grid=(GRID,),
    in_specs=[_row_spec(64), _row_spec(1), _row_spec(64)],
    out_specs=[_row_spec(64), _full_spec((2, 64))],
    out_shape=[jax.ShapeDtypeStruct((N, 64), jnp.float32),
               jax.ShapeDtypeStruct((2, 64), jnp.float32)],
)


def _final_body(x_ref, base_ref, m_ref, v_ref, g_ref, b_ref, w1_ref, b1_ref,
                w2_ref, b2_ref, y_ref):
    emb = base_ref[...] + _bn(x_ref[...], m_ref, v_ref, g_ref, b_ref)
    h = _elu(jnp.dot(emb, w1_ref[...], preferred_element_type=jnp.float32)
             + b1_ref[...])
    y_ref[...] = (jnp.dot(h, w2_ref[...], preferred_element_type=jnp.float32)
                  + b2_ref[...])


_final_call = pl.pallas_call(
    _final_body,
    grid=(GRID,),
    in_specs=[_row_spec(64), _row_spec(64), _full_spec((1, 64)),
              _full_spec((1, 64)), _full_spec((1, 64)), _full_spec((1, 64)),
              _full_spec((64, 32)), _full_spec((1, 32)),
              _full_spec((32, 1)), _full_spec((1, 1))],
    out_specs=[_row_spec(1)],
    out_shape=[jax.ShapeDtypeStruct((N, 1), jnp.float32)],
)


def _moments(s):
    m = s[0:1] / N
    v = s[1:2] / N - m * m
    return m, v


def kernel(x_cont, x_cat, edge_index, edge_attr, batch, norm, tab_chrg,
           tab_pdg, W_cont, b_cont, W_cat, b_cat, W_enc, b_enc, g_all, b_all,
           W_msg0, b_msg0, g_bn0, b_bn0, W_msg1, b_msg1, g_bn1, b_bn1,
           W_out1, b_out1, W_out2, b_out2):
    row = jnp.concatenate([edge_index[0], jnp.zeros((E_PAD - E,), jnp.int32)])
    col = jnp.concatenate([edge_index[1], jnp.full((E_PAD - E,), N, jnp.int32)])
    row2d = row.reshape(CR, CH)
    rows_st = jnp.concatenate([row2d, row2d + N], axis=0)
    col2d = col.reshape(CR, CH)
    ones16 = jnp.ones((CH, 16), jnp.float32)
    zero16 = jnp.zeros((NPAD, 16), jnp.float32)
    zero32 = jnp.zeros((NPAD, HID // NC), jnp.float32)

    cnt_pad = _sc_count(col2d, ones16, zero16)
    cnt = cnt_pad[:N, 0:1] + cnt_pad[NPAD:NPAD + N, 0:1]

    def seg(a):
        a_st = jnp.concatenate([a[:, :HID // NC], a[:, HID // NC:]], axis=0)
        sp = _sc_segsum(a_st, rows_st, col2d, zero32)
        return jnp.concatenate([sp[:N], sp[NPAD:NPAD + N]], axis=1)

    z, s = _enc_call(x_cont, x_cat, norm.reshape(1, 6), tab_chrg, tab_pdg,
                     W_cont, b_cont.reshape(1, 32), W_cat, b_cat.reshape(1, 32),
                     W_enc, b_enc.reshape(1, 64))
    m, v = _moments(s)
    emb, a0, bb0 = _proj_call(z, m, v, g_all.reshape(1, 64),
                              b_all.reshape(1, 64), W_msg0[:HID], W_msg0[HID:],
                              b_msg0.reshape(1, 64))
    s0 = seg(a0)
    out0, s = _combine_call(s0, cnt, bb0)
    m, v = _moments(s)
    emb1, a1, bb1 = _proj_res_call(out0, emb, m, v, g_bn0.reshape(1, 64),
                                   b_bn0.reshape(1, 64), W_msg1[:HID],
                                   W_msg1[HID:], b_msg1.reshape(1, 64))
    s1 = seg(a1)
    out1, s = _combine_call(s1, cnt, bb1)
    m, v = _moments(s)
    (y,) = _final_call(out1, emb1, m, v, g_bn1.reshape(1, 64),
                       b_bn1.reshape(1, 64), W_out1, b_out1.reshape(1, 32),
                       W_out2, b_out2.reshape(1, 1))
    return y[:, 0]


# speedup vs baseline: 8.1990x; 1.2000x over previous
import functools

import jax
import jax.numpy as jnp
from jax import lax
from jax.experimental import pallas as pl
from jax.experimental.pallas import tpu as pltpu
from jax.experimental.pallas import tpu_sc as plsc

N = 50000
E = 800000
HID = 64
PDG_LIST = [1, 2, 11, 13, 22, 130, 211]

NC = 2
NS = 16
CH = 128
CR = 6336
E_PAD = CR * CH
NPAD = 50048
RPN = NPAD // NS
G = 2
NSC = CR // NS // G
G2 = 11
NSC2 = CR // (NC * NS) // G2

RB = 5000
GRID = N // RB

def _elu(x):
    return jnp.where(x > 0, x, jnp.exp(jnp.minimum(x, 0.0)) - 1.0)


@functools.cache
def _sc_kernels():
    mesh = plsc.VectorSubcoreMesh(core_axis_name="c", subcore_axis_name="s",
                                  num_cores=NC, num_subcores=NS)

    @functools.partial(
        pl.kernel, mesh=mesh,
        out_type=jax.ShapeDtypeStruct((NC * NPAD, 16), jnp.float32),
        scratch_types=[
            pltpu.VMEM((2, G2, CH), jnp.int32),
            pltpu.VMEM((CH, 16), jnp.float32),
            pltpu.VMEM_SHARED((NPAD, 16), jnp.float32),
            pltpu.SemaphoreType.DMA,
            pltpu.SemaphoreType.DMA,
            pltpu.SemaphoreType.DMA,
            pltpu.SemaphoreType.DMA,
        ],
        compiler_params=pltpu.CompilerParams(use_tc_tiling_on_sc=False),
    )
    def sc_count(col_hbm, ones_hbm, zero_hbm, cnt_hbm, colb, onesv, acc,
                 isem0, isem1, ssem0, ssem1):
        c = lax.axis_index("c")
        s = lax.axis_index("s")
        isem = (isem0, isem1)
        ssem = (ssem0, ssem1)
        sub = (c * NS + s) * (G2 * NSC2)
        pltpu.sync_copy(zero_hbm.at[pl.ds(s * RPN, RPN)],
                        acc.at[pl.ds(s * RPN, RPN)])
        pltpu.sync_copy(ones_hbm, onesv)
        plsc.subcore_barrier()

        def load(u, b):
            pltpu.async_copy(col_hbm.at[pl.ds(sub + u * G2, G2)],
                             colb.at[b], isem[b])

        def wait_load(b):
            pltpu.make_async_copy(col_hbm.at[pl.ds(sub, G2)],
                                  colb.at[b], isem[b]).wait()

        def fire_scatter(b):
            for j in range(G2):
                pltpu.async_copy(onesv, acc.at[colb.at[b, j]], ssem[b],
                                 add=True)

        def wait_scatter(b):
            for j in range(G2):
                pltpu.make_async_copy(onesv, acc.at[colb.at[b, j]],
                                      ssem[b]).wait()

        load(0, 0)
        load(1, 1)

        def outer(w, _):
            for off, b in ((0, 0), (1, 1)):
                u = 2 * w + off
                wait_load(b)
                fire_scatter(b)
                wait_scatter(b)

                @pl.when(u + 2 < NSC2)
                def _():
                    load(u + 2, b)
            return 0

        lax.fori_loop(0, NSC2 // 2, outer, 0)
        plsc.subcore_barrier()
        pltpu.sync_copy(acc.at[pl.ds(s * RPN, RPN)],
                        cnt_hbm.at[pl.ds(c * NPAD + s * RPN, RPN)])

    @functools.partial(
        pl.kernel, mesh=mesh,
        out_type=jax.ShapeDtypeStruct((NC * NPAD, HID // NC), jnp.float32),
        scratch_types=[
            pltpu.VMEM((3, G, CH), jnp.int32),
            pltpu.VMEM((3, G, CH), jnp.int32),
            pltpu.VMEM((3, G, CH, HID // NC), jnp.float32),
            pltpu.VMEM_SHARED((NPAD, HID // NC), jnp.float32),
        ] + [pltpu.SemaphoreType.DMA] * 9,
        compiler_params=pltpu.CompilerParams(use_tc_tiling_on_sc=False),
    )
    def sc_segsum(a_hbm, row_hbm, col_hbm, zero_hbm, out_hbm,
                  rowb, colb, gb, acc, *sems):
        c = lax.axis_index("c")
        s = lax.axis_index("s")
        isem = sems[0:3]
        gsem = sems[3:6]
        ssem = sems[6:9]
        sub = s * (G * NSC)
        pltpu.sync_copy(zero_hbm.at[pl.ds(s * RPN, RPN)],
                        acc.at[pl.ds(s * RPN, RPN)])
        plsc.subcore_barrier()

        def load(u, b):
            pltpu.async_copy(row_hbm.at[pl.ds(sub + u * G, G)],
                             rowb.at[b], isem[b])
            pltpu.async_copy(col_hbm.at[pl.ds(sub + u * G, G)],
                             colb.at[b], isem[b])

        def wait_load(b):
            pltpu.make_async_copy(row_hbm.at[pl.ds(sub, G)],
                                  rowb.at[b], isem[b]).wait()
            pltpu.make_async_copy(col_hbm.at[pl.ds(sub, G)],
                                  colb.at[b], isem[b]).wait()

        def transform(b):
            for j in range(G):
                for q in range(CH // 16):
                    sl = pl.ds(q * 16, 16)
                    rowb[b, j, sl] = rowb[b, j, sl] * 2 + c

        def fire_gather(b):
            for j in range(G):
                pltpu.async_copy(a_hbm.at[rowb.at[b, j]], gb.at[b, j],
                                 gsem[b])

        def wait_gather(b):
            for j in range(G):
                pltpu.make_async_copy(a_hbm.at[rowb.at[b, j]], gb.at[b, j],
                                      gsem[b]).wait()

        def fire_scatter(b):
            for j in range(G):
                pltpu.async_copy(gb.at[b, j], acc.at[colb.at[b, j]], ssem[b],
                                 add=True)

        def wait_scatter(b):
            for j in range(G):
                pltpu.make_async_copy(gb.at[b, j], acc.at[colb.at[b, j]],
                                      ssem[b]).wait()

        load(0, 0)
        wait_load(0)
        transform(0)
        fire_gather(0)
        load(1, 1)

        def outer(w, _):
            for b in range(3):
                u = 3 * w + b
                b1 = (b + 1) % 3
                b2 = (b + 2) % 3

                @pl.when(u + 1 < NSC)
                def _():
                    wait_load(b1)
                    transform(b1)
                    fire_gather(b1)

                wait_gather(b)
                fire_scatter(b)

                @pl.when(u >= 1)
                def _():
                    wait_scatter(b2)

                @pl.when(u + 2 < NSC)
                def _():
                    load(u + 2, b2)
            return 0

        lax.fori_loop(0, NSC // 3, outer, 0)
        wait_scatter((NSC - 1) % 3)
        plsc.subcore_barrier()
        pltpu.sync_copy(acc.at[pl.ds(s * RPN, RPN)],
                        out_hbm.at[pl.ds(c * NPAD + s * RPN, RPN)])

    return sc_count, sc_segsum


def _sc_count(col2d, ones16, zero16):
    return _sc_kernels()[0](col2d, ones16, zero16)


def _sc_segsum(a_r, row2d, col2d, zero32):
    return _sc_kernels()[1](a_r, row2d, col2d, zero32)


def _row_spec(cols):
    return pl.BlockSpec((RB, cols), lambda i: (i, 0))


def _full_spec(shape):
    return pl.BlockSpec(shape, lambda i: tuple(0 for _ in shape))


def _acc_sums(i, x, s_ref):
    @pl.when(i == 0)
    def _():
        s_ref[...] = jnp.zeros_like(s_ref)
    s1 = jnp.sum(x, axis=0, keepdims=True)
    s2 = jnp.sum(x * x, axis=0, keepdims=True)
    s_ref[...] += jnp.concatenate([s1, s2], axis=0)


def _enc_body(xc_ref, xcat_ref, norm_ref, tc_ref, tp_ref, wc_ref, bc_ref,
              wk_ref, bk_ref, we_ref, be_ref, z_ref, s_ref):
    i = pl.program_id(0)
    xc = xc_ref[...] * norm_ref[...]
    emb_cont = _elu(jnp.dot(xc, wc_ref[...], preferred_element_type=jnp.float32)
                    + bc_ref[...])
    chrg = xcat_ref[:, 1:2] + 1
    oh_c = (chrg == lax.broadcasted_iota(jnp.int32, (RB, 3), 1)).astype(jnp.float32)
    emb_chrg = jnp.dot(oh_c, tc_ref[...], preferred_element_type=jnp.float32)
    r = jnp.abs(xcat_ref[:, 0:1])
    for k, p in enumerate(PDG_LIST):
        r = jnp.where(r == p, jnp.full_like(r, k), r)
    oh_p = (r == lax.broadcasted_iota(jnp.int32, (RB, 7), 1)).astype(jnp.float32)
    emb_pdg = jnp.dot(oh_p, tp_ref[...], preferred_element_type=jnp.float32)
    cat = jnp.concatenate([emb_chrg, emb_pdg], axis=1)
    emb_cat = _elu(jnp.dot(cat, wk_ref[...], preferred_element_type=jnp.float32)
                   + bk_ref[...])
    both = jnp.concatenate([emb_cat, emb_cont], axis=1)
    z = _elu(jnp.dot(both, we_ref[...], preferred_element_type=jnp.float32)
             + be_ref[...])
    z_ref[...] = z
    _acc_sums(i, z, s_ref)


_enc_call = pl.pallas_call(
    _enc_body,
    grid=(GRID,),
    in_specs=[
        _row_spec(6), _row_spec(2), _full_spec((1, 6)),
        _full_spec((3, 16)), _full_spec((7, 16)),
        _full_spec((6, 32)), _full_spec((1, 32)),
        _full_spec((32, 32)), _full_spec((1, 32)),
        _full_spec((64, 64)), _full_spec((1, 64)),
    ],
    out_specs=[_row_spec(64), _full_spec((2, 64))],
    out_shape=[jax.ShapeDtypeStruct((N, 64), jnp.float32),
               jax.ShapeDtypeStruct((2, 64), jnp.float32)],
)


def _bn(x, m_ref, v_ref, g_ref, b_ref):
    return (g_ref[...] * (x - m_ref[...]) * lax.rsqrt(v_ref[...] + 1e-5)
            + b_ref[...])


def _proj_body(x_ref, m_ref, v_ref, g_ref, b_ref, wt_ref, wb_ref, bm_ref,
               emb_ref, a_ref, bb_ref):
    emb = _bn(x_ref[...], m_ref, v_ref, g_ref, b_ref)
    emb_ref[...] = emb
    a_ref[...] = jnp.dot(emb, wt_ref[...], preferred_element_type=jnp.float32)
    bb_ref[...] = (jnp.dot(emb, wb_ref[...], preferred_element_type=jnp.float32)
                   + bm_ref[...])


def _proj_res_body(x_ref, base_ref, m_ref, v_ref, g_ref, b_ref, wt_ref, wb_ref,
                   bm_ref, emb_ref, a_ref, bb_ref):
    emb = base_ref[...] + _bn(x_ref[...], m_ref, v_ref, g_ref, b_ref)
    emb_ref[...] = emb
    a_ref[...] = jnp.dot(emb, wt_ref[...], preferred_element_type=jnp.float32)
    bb_ref[...] = (jnp.dot(emb, wb_ref[...], preferred_element_type=jnp.float32)
                   + bm_ref[...])


_proj_specs = dict(
    grid=(GRID,),
    out_specs=[_row_spec(64), _row_spec(64), _row_spec(64)],
    out_shape=[jax.ShapeDtypeStruct((N, 64), jnp.float32)] * 3,
)
_proj_call = pl.pallas_call(
    _proj_body,
    in_specs=[_row_spec(64), _full_spec((1, 64)), _full_spec((1, 64)),
              _full_spec((1, 64)), _full_spec((1, 64)),
              _full_spec((64, 64)), _full_spec((64, 64)), _full_spec((1, 64))],
    **_proj_specs,
)
_proj_res_call = pl.pallas_call(
    _proj_res_body,
    in_specs=[_row_spec(64), _row_spec(64), _full_spec((1, 64)),
              _full_spec((1, 64)), _full_spec((1, 64)), _full_spec((1, 64)),
              _full_spec((64, 64)), _full_spec((64, 64)), _full_spec((1, 64))],
    **_proj_specs,
)


def _combine_body(s_ref, cnt_ref, bb_ref, o_ref, sum_ref):
    i = pl.program_id(0)
    cnt = cnt_ref[...]
    o = jnp.where(cnt > 0,
                  s_ref[...] / jnp.maximum(cnt, 1.0) + bb_ref[...],
                  0.0)
    o_ref[...] = o
    _acc_sums(i, o, sum_ref)


_combine_call = pl.pallas_call(
    _combine_body,
    grid=(GRID,),
    in_specs=[_row_spec(64), _row_spec(1), _row_spec(64)],
    out_specs=[_row_spec(64), _full_spec((2, 64))],
    out_shape=[jax.ShapeDtypeStruct((N, 64), jnp.float32),
               jax.ShapeDtypeStruct((2, 64), jnp.float32)],
)


def _final_body(x_ref, base_ref, m_ref, v_ref, g_ref, b_ref, w1_ref, b1_ref,
                w2_ref, b2_ref, y_ref):
    emb = base_ref[...] + _bn(x_ref[...], m_ref, v_ref, g_ref, b_ref)
    h = _elu(jnp.dot(emb, w1_ref[...], preferred_element_type=jnp.float32)
             + b1_ref[...])
    y_ref[...] = (jnp.dot(h, w2_ref[...], preferred_element_type=jnp.float32)
                  + b2_ref[...])


_final_call = pl.pallas_call(
    _final_body,
    grid=(GRID,),
    in_specs=[_row_spec(64), _row_spec(64), _full_spec((1, 64)),
              _full_spec((1, 64)), _full_spec((1, 64)), _full_spec((1, 64)),
              _full_spec((64, 32)), _full_spec((1, 32)),
              _full_spec((32, 1)), _full_spec((1, 1))],
    out_specs=[_row_spec(1)],
    out_shape=[jax.ShapeDtypeStruct((N, 1), jnp.float32)],
)


def _moments(s):
    m = s[0:1] / N
    v = s[1:2] / N - m * m
    return m, v


def kernel(x_cont, x_cat, edge_index, edge_attr, batch, norm, tab_chrg,
           tab_pdg, W_cont, b_cont, W_cat, b_cat, W_enc, b_enc, g_all, b_all,
           W_msg0, b_msg0, g_bn0, b_bn0, W_msg1, b_msg1, g_bn1, b_bn1,
           W_out1, b_out1, W_out2, b_out2):
    row = jnp.concatenate([edge_index[0], jnp.zeros((E_PAD - E,), jnp.int32)])
    col = jnp.concatenate([edge_index[1], jnp.full((E_PAD - E,), N, jnp.int32)])
    row2d = row.reshape(CR, CH)
    col2d = col.reshape(CR, CH)
    ones16 = jnp.ones((CH, 16), jnp.float32)
    zero16 = jnp.zeros((NPAD, 16), jnp.float32)
    zero32 = jnp.zeros((NPAD, HID // NC), jnp.float32)

    cnt_pad = _sc_count(col2d, ones16, zero16)
    cnt = cnt_pad[:N, 0:1] + cnt_pad[NPAD:NPAD + N, 0:1]

    def seg(a):
        sp = _sc_segsum(a.reshape(NC * N, HID // NC), row2d, col2d, zero32)
        return jnp.concatenate([sp[:N], sp[NPAD:NPAD + N]], axis=1)

    z, s = _enc_call(x_cont, x_cat, norm.reshape(1, 6), tab_chrg, tab_pdg,
                     W_cont, b_cont.reshape(1, 32), W_cat, b_cat.reshape(1, 32),
                     W_enc, b_enc.reshape(1, 64))
    m, v = _moments(s)
    emb, a0, bb0 = _proj_call(z, m, v, g_all.reshape(1, 64),
                              b_all.reshape(1, 64), W_msg0[:HID], W_msg0[HID:],
                              b_msg0.reshape(1, 64))
    s0 = seg(a0)
    out0, s = _combine_call(s0, cnt, bb0)
    m, v = _moments(s)
    emb1, a1, bb1 = _proj_res_call(out0, emb, m, v, g_bn0.reshape(1, 64),
                                   b_bn0.reshape(1, 64), W_msg1[:HID],
                                   W_msg1[HID:], b_msg1.reshape(1, 64))
    s1 = seg(a1)
    out1, s = _combine_call(s1, cnt, bb1)
    m, v = _moments(s)
    (y,) = _final_call(out1, emb1, m, v, g_bn1.reshape(1, 64),
                       b_bn1.reshape(1, 64), W_out1, b_out1.reshape(1, 32),
                       W_out2, b_out2.reshape(1, 1))
    return y[:, 0]


# speedup vs baseline: 8.9422x; 1.0906x over previous
import functools

import jax
import jax.numpy as jnp
from jax import lax
from jax.experimental import pallas as pl
from jax.experimental.pallas import tpu as pltpu
from jax.experimental.pallas import tpu_sc as plsc

N = 50000
E = 800000
HID = 64
PDG_LIST = [1, 2, 11, 13, 22, 130, 211]

NC = 2
NS = 16
CH = 128
CR = 6336
E_PAD = CR * CH
NPAD = 50048
RPN = NPAD // NS
G = 2
NSC = CR // NS // G
G2 = 11
NSC2 = CR // (NC * NS) // G2

RB = 5000
GRID = N // RB

def _elu(x):
    return jnp.where(x > 0, x, jnp.exp(jnp.minimum(x, 0.0)) - 1.0)


@functools.cache
def _sc_kernels():
    mesh = plsc.VectorSubcoreMesh(core_axis_name="c", subcore_axis_name="s",
                                  num_cores=NC, num_subcores=NS)

    @functools.partial(
        pl.kernel, mesh=mesh,
        out_type=jax.ShapeDtypeStruct((NC, NPAD, 16), jnp.float32),
        scratch_types=[
            pltpu.VMEM((2, G2, CH), jnp.int32),
            pltpu.VMEM((CH, 16), jnp.float32),
            pltpu.VMEM_SHARED((NPAD, 16), jnp.float32),
            pltpu.SemaphoreType.DMA,
            pltpu.SemaphoreType.DMA,
            pltpu.SemaphoreType.DMA,
            pltpu.SemaphoreType.DMA,
        ],
        compiler_params=pltpu.CompilerParams(use_tc_tiling_on_sc=False),
    )
    def sc_count(col_hbm, ones_hbm, zero_hbm, cnt_hbm, colb, onesv, acc,
                 isem0, isem1, ssem0, ssem1):
        c = lax.axis_index("c")
        s = lax.axis_index("s")
        isem = (isem0, isem1)
        ssem = (ssem0, ssem1)
        sub = (c * NS + s) * (G2 * NSC2)
        pltpu.sync_copy(zero_hbm.at[pl.ds(s * RPN, RPN)],
                        acc.at[pl.ds(s * RPN, RPN)])
        pltpu.sync_copy(ones_hbm, onesv)
        plsc.subcore_barrier()

        def load(u, b):
            pltpu.async_copy(col_hbm.at[pl.ds(sub + u * G2, G2)],
                             colb.at[b], isem[b])

        def wait_load(b):
            pltpu.make_async_copy(col_hbm.at[pl.ds(sub, G2)],
                                  colb.at[b], isem[b]).wait()

        def fire_scatter(b):
            for j in range(G2):
                pltpu.async_copy(onesv, acc.at[colb.at[b, j]], ssem[b],
                                 add=True)

        def wait_scatter(b):
            for j in range(G2):
                pltpu.make_async_copy(onesv, acc.at[colb.at[b, j]],
                                      ssem[b]).wait()

        load(0, 0)
        load(1, 1)

        def outer(w, _):
            for off, b in ((0, 0), (1, 1)):
                u = 2 * w + off
                wait_load(b)
                fire_scatter(b)
                wait_scatter(b)

                @pl.when(u + 2 < NSC2)
                def _():
                    load(u + 2, b)
            return 0

        lax.fori_loop(0, NSC2 // 2, outer, 0)
        plsc.subcore_barrier()
        pltpu.sync_copy(acc.at[pl.ds(s * RPN, RPN)],
                        cnt_hbm.at[c, pl.ds(s * RPN, RPN)])

    @functools.partial(
        pl.kernel, mesh=mesh,
        out_type=jax.ShapeDtypeStruct((NC, NPAD, HID // NC), jnp.float32),
        scratch_types=[
            pltpu.VMEM((3, G, CH), jnp.int32),
            pltpu.VMEM((3, G, CH), jnp.int32),
            pltpu.VMEM((3, G, CH, HID // NC), jnp.float32),
            pltpu.VMEM_SHARED((NPAD, HID // NC), jnp.float32),
        ] + [pltpu.SemaphoreType.DMA] * 9,
        compiler_params=pltpu.CompilerParams(use_tc_tiling_on_sc=False),
    )
    def sc_segsum(a_hbm, row_hbm, col_hbm, zero_hbm, out_hbm,
                  rowb, colb, gb, acc, *sems):
        c = lax.axis_index("c")
        s = lax.axis_index("s")
        isem = sems[0:3]
        gsem = sems[3:6]
        ssem = sems[6:9]
        sub = s * (G * NSC)
        pltpu.sync_copy(zero_hbm.at[pl.ds(s * RPN, RPN)],
                        acc.at[pl.ds(s * RPN, RPN)])
        plsc.subcore_barrier()

        def load(u, b):
            pltpu.async_copy(row_hbm.at[pl.ds(sub + u * G, G)],
                             rowb.at[b], isem[b])
            pltpu.async_copy(col_hbm.at[pl.ds(sub + u * G, G)],
                             colb.at[b], isem[b])

        def wait_load(b):
            pltpu.make_async_copy(row_hbm.at[pl.ds(sub, G)],
                                  rowb.at[b], isem[b]).wait()
            pltpu.make_async_copy(col_hbm.at[pl.ds(sub, G)],
                                  colb.at[b], isem[b]).wait()

        def transform(b):
            for j in range(G):
                for q in range(CH // 16):
                    sl = pl.ds(q * 16, 16)
                    rowb[b, j, sl] = rowb[b, j, sl] * 2 + c

        def fire_gather(b):
            for j in range(G):
                pltpu.async_copy(a_hbm.at[rowb.at[b, j]], gb.at[b, j],
                                 gsem[b])

        def wait_gather(b):
            for j in range(G):
                pltpu.make_async_copy(a_hbm.at[rowb.at[b, j]], gb.at[b, j],
                                      gsem[b]).wait()

        def fire_scatter(b):
            for j in range(G):
                pltpu.async_copy(gb.at[b, j], acc.at[colb.at[b, j]], ssem[b],
                                 add=True)

        def wait_scatter(b):
            for j in range(G):
                pltpu.make_async_copy(gb.at[b, j], acc.at[colb.at[b, j]],
                                      ssem[b]).wait()

        load(0, 0)
        wait_load(0)
        transform(0)
        fire_gather(0)
        load(1, 1)

        def outer(w, _):
            for b in range(3):
                u = 3 * w + b
                b1 = (b + 1) % 3
                b2 = (b + 2) % 3

                @pl.when(u + 1 < NSC)
                def _():
                    wait_load(b1)
                    transform(b1)
                    fire_gather(b1)

                wait_gather(b)
                fire_scatter(b)

                @pl.when(u >= 1)
                def _():
                    wait_scatter(b2)

                @pl.when(u + 2 < NSC)
                def _():
                    load(u + 2, b2)
            return 0

        lax.fori_loop(0, NSC // 3, outer, 0)
        wait_scatter((NSC - 1) % 3)
        plsc.subcore_barrier()
        pltpu.sync_copy(acc.at[pl.ds(s * RPN, RPN)],
                        out_hbm.at[c, pl.ds(s * RPN, RPN)])

    return sc_count, sc_segsum


def _sc_count(col2d, ones16, zero16):
    return _sc_kernels()[0](col2d, ones16, zero16)


def _sc_segsum(a_r, row2d, col2d, zero32):
    return _sc_kernels()[1](a_r, row2d, col2d, zero32)


def _row_spec(cols):
    return pl.BlockSpec((RB, cols), lambda i: (i, 0))


def _full_spec(shape):
    return pl.BlockSpec(shape, lambda i: tuple(0 for _ in shape))


def _acc_sums(i, x, s_ref):
    @pl.when(i == 0)
    def _():
        s_ref[...] = jnp.zeros_like(s_ref)
    s1 = jnp.sum(x, axis=0, keepdims=True)
    s2 = jnp.sum(x * x, axis=0, keepdims=True)
    s_ref[...] += jnp.concatenate([s1, s2], axis=0)


def _enc_body(xc_ref, xcat_ref, norm_ref, tc_ref, tp_ref, wc_ref, bc_ref,
              wk_ref, bk_ref, we_ref, be_ref, z_ref, s_ref):
    i = pl.program_id(0)
    xc = xc_ref[...] * norm_ref[...]
    emb_cont = _elu(jnp.dot(xc, wc_ref[...], preferred_element_type=jnp.float32)
                    + bc_ref[...])
    chrg = xcat_ref[:, 1:2] + 1
    oh_c = (chrg == lax.broadcasted_iota(jnp.int32, (RB, 3), 1)).astype(jnp.float32)
    emb_chrg = jnp.dot(oh_c, tc_ref[...], preferred_element_type=jnp.float32)
    r = jnp.abs(xcat_ref[:, 0:1])
    for k, p in enumerate(PDG_LIST):
        r = jnp.where(r == p, jnp.full_like(r, k), r)
    oh_p = (r == lax.broadcasted_iota(jnp.int32, (RB, 7), 1)).astype(jnp.float32)
    emb_pdg = jnp.dot(oh_p, tp_ref[...], preferred_element_type=jnp.float32)
    cat = jnp.concatenate([emb_chrg, emb_pdg], axis=1)
    emb_cat = _elu(jnp.dot(cat, wk_ref[...], preferred_element_type=jnp.float32)
                   + bk_ref[...])
    both = jnp.concatenate([emb_cat, emb_cont], axis=1)
    z = _elu(jnp.dot(both, we_ref[...], preferred_element_type=jnp.float32)
             + be_ref[...])
    z_ref[...] = z
    _acc_sums(i, z, s_ref)


_enc_call = pl.pallas_call(
    _enc_body,
    grid=(GRID,),
    in_specs=[
        _row_spec(6), _row_spec(2), _full_spec((1, 6)),
        _full_spec((3, 16)), _full_spec((7, 16)),
        _full_spec((6, 32)), _full_spec((1, 32)),
        _full_spec((32, 32)), _full_spec((1, 32)),
        _full_spec((64, 64)), _full_spec((1, 64)),
    ],
    out_specs=[_row_spec(64), _full_spec((2, 64))],
    out_shape=[jax.ShapeDtypeStruct((N, 64), jnp.float32),
               jax.ShapeDtypeStruct((2, 64), jnp.float32)],
)


def _bn(x, m_ref, v_ref, g_ref, b_ref):
    return (g_ref[...] * (x - m_ref[...]) * lax.rsqrt(v_ref[...] + 1e-5)
            + b_ref[...])


def _proj_body(x_ref, m_ref, v_ref, g_ref, b_ref, wt_ref, wb_ref, bm_ref,
               emb_ref, a_ref, bb_ref):
    emb = _bn(x_ref[...], m_ref, v_ref, g_ref, b_ref)
    emb_ref[...] = emb
    a_ref[...] = jnp.dot(emb, wt_ref[...], preferred_element_type=jnp.float32)
    bb_ref[...] = (jnp.dot(emb, wb_ref[...], preferred_element_type=jnp.float32)
                   + bm_ref[...])


def _proj_res_body(x_ref, base_ref, m_ref, v_ref, g_ref, b_ref, wt_ref, wb_ref,
                   bm_ref, emb_ref, a_ref, bb_ref):
    emb = base_ref[...] + _bn(x_ref[...], m_ref, v_ref, g_ref, b_ref)
    emb_ref[...] = emb
    a_ref[...] = jnp.dot(emb, wt_ref[...], preferred_element_type=jnp.float32)
    bb_ref[...] = (jnp.dot(emb, wb_ref[...], preferred_element_type=jnp.float32)
                   + bm_ref[...])


_proj_specs = dict(
    grid=(GRID,),
    out_specs=[_row_spec(64), _row_spec(64), _row_spec(64)],
    out_shape=[jax.ShapeDtypeStruct((N, 64), jnp.float32)] * 3,
)
_proj_call = pl.pallas_call(
    _proj_body,
    in_specs=[_row_spec(64), _full_spec((1, 64)), _full_spec((1, 64)),
              _full_spec((1, 64)), _full_spec((1, 64)),
              _full_spec((64, 64)), _full_spec((64, 64)), _full_spec((1, 64))],
    **_proj_specs,
)
_proj_res_call = pl.pallas_call(
    _proj_res_body,
    in_specs=[_row_spec(64), _row_spec(64), _full_spec((1, 64)),
              _full_spec((1, 64)), _full_spec((1, 64)), _full_spec((1, 64)),
              _full_spec((64, 64)), _full_spec((64, 64)), _full_spec((1, 64))],
    **_proj_specs,
)


def _combine_body(s0_ref, s1_ref, c0_ref, c1_ref, bb_ref, o_ref, sum_ref):
    i = pl.program_id(0)
    cnt = c0_ref[0, :, 0:1] + c1_ref[0, :, 0:1]
    sfull = jnp.concatenate([s0_ref[0], s1_ref[0]], axis=1)
    o = jnp.where(cnt > 0,
                  sfull / jnp.maximum(cnt, 1.0) + bb_ref[...],
                  0.0)
    o_ref[...] = o
    _acc_sums(i, o, sum_ref)


_combine_call = pl.pallas_call(
    _combine_body,
    grid=(GRID,),
    in_specs=[
        pl.BlockSpec((1, RB, 32), lambda i: (0, i, 0)),
        pl.BlockSpec((1, RB, 32), lambda i: (1, i, 0)),
        pl.BlockSpec((1, RB, 16), lambda i: (0, i, 0)),
        pl.BlockSpec((1, RB, 16), lambda i: (1, i, 0)),
        _row_spec(64),
    ],
    out_specs=[_row_spec(64), _full_spec((2, 64))],
    out_shape=[jax.ShapeDtypeStruct((N, 64), jnp.float32),
               jax.ShapeDtypeStruct((2, 64), jnp.float32)],
)


def _final_body(x_ref, base_ref, m_ref, v_ref, g_ref, b_ref, w1_ref, b1_ref,
                w2_ref, b2_ref, y_ref):
    emb = base_ref[...] + _bn(x_ref[...], m_ref, v_ref, g_ref, b_ref)
    h = _elu(jnp.dot(emb, w1_ref[...], preferred_element_type=jnp.float32)
             + b1_ref[...])
    y_ref[...] = (jnp.dot(h, w2_ref[...], preferred_element_type=jnp.float32)
                  + b2_ref[...])


_final_call = pl.pallas_call(
    _final_body,
    grid=(GRID,),
    in_specs=[_row_spec(64), _row_spec(64), _full_spec((1, 64)),
              _full_spec((1, 64)), _full_spec((1, 64)), _full_spec((1, 64)),
              _full_spec((64, 32)), _full_spec((1, 32)),
              _full_spec((32, 1)), _full_spec((1, 1))],
    out_specs=[_row_spec(1)],
    out_shape=[jax.ShapeDtypeStruct((N, 1), jnp.float32)],
)


def _moments(s):
    m = s[0:1] / N
    v = s[1:2] / N - m * m
    return m, v


def kernel(x_cont, x_cat, edge_index, edge_attr, batch, norm, tab_chrg,
           tab_pdg, W_cont, b_cont, W_cat, b_cat, W_enc, b_enc, g_all, b_all,
           W_msg0, b_msg0, g_bn0, b_bn0, W_msg1, b_msg1, g_bn1, b_bn1,
           W_out1, b_out1, W_out2, b_out2):
    row = jnp.concatenate([edge_index[0], jnp.zeros((E_PAD - E,), jnp.int32)])
    col = jnp.concatenate([edge_index[1], jnp.full((E_PAD - E,), N, jnp.int32)])
    row2d, col2d = jax.lax.optimization_barrier(
        (row.reshape(CR, CH), col.reshape(CR, CH)))
    ones16 = jnp.ones((CH, 16), jnp.float32)
    zero16 = jnp.zeros((NPAD, 16), jnp.float32)
    zero32 = jnp.zeros((NPAD, HID // NC), jnp.float32)

    cnt_pad = _sc_count(col2d, ones16, zero16)

    def seg(a):
        return _sc_segsum(a.reshape(NC * N, HID // NC), row2d, col2d, zero32)

    z, s = _enc_call(x_cont, x_cat, norm.reshape(1, 6), tab_chrg, tab_pdg,
                     W_cont, b_cont.reshape(1, 32), W_cat, b_cat.reshape(1, 32),
                     W_enc, b_enc.reshape(1, 64))
    m, v = _moments(s)
    emb, a0, bb0 = _proj_call(z, m, v, g_all.reshape(1, 64),
                              b_all.reshape(1, 64), W_msg0[:HID], W_msg0[HID:],
                              b_msg0.reshape(1, 64))
    s0 = seg(a0)
    out0, s = _combine_call(s0, s0, cnt_pad, cnt_pad, bb0)
    m, v = _moments(s)
    emb1, a1, bb1 = _proj_res_call(out0, emb, m, v, g_bn0.reshape(1, 64),
                                   b_bn0.reshape(1, 64), W_msg1[:HID],
                                   W_msg1[HID:], b_msg1.reshape(1, 64))
    s1 = seg(a1)
    out1, s = _combine_call(s1, s1, cnt_pad, cnt_pad, bb1)
    m, v = _moments(s)
    (y,) = _final_call(out1, emb1, m, v, g_bn1.reshape(1, 64),
                       b_bn1.reshape(1, 64), W_out1, b_out1.reshape(1, 32),
                       W_out2, b_out2.reshape(1, 1))
    return y[:, 0]
